# baseline (speedup 1.0000x reference)
import numpy as np
import jax
import jax.numpy as jnp
from jax import lax
from jax.experimental import pallas as pl
from jax.experimental.pallas import tpu as pltpu
from jax.experimental.pallas import tpu_sc as plsc

_NUM_FIELDS = 26
_VOCAB = 100000
_EMBED = 16
_BATCH = 16384
_B = _BATCH * _NUM_FIELDS
_NC = 2
_NS = 16
_NW = _NC * _NS
_BPW = _B // _NW
_CHUNK = 1664
_NCHUNK = _BPW // _CHUNK
_NBUF = 2
_PERIOD = 208
_PVREGS = _PERIOD // 16
_NPERIOD = _BPW // _PERIOD

_OFFS_TILED = np.tile(
    np.array((0, *np.cumsum([_VOCAB] * _NUM_FIELDS)[:-1]), dtype=np.int32),
    _PERIOD // _NUM_FIELDS,
)


def _sc_body(x_hbm, offs_hbm, table_hbm, out_hbm, idx_v, offs_v, rows_v,
             gsem, ssem):
    wid = lax.axis_index("s") * _NC + lax.axis_index("c")
    base = wid * _BPW

    pltpu.sync_copy(offs_hbm, offs_v)
    pltpu.sync_copy(x_hbm.at[pl.ds(base, _BPW)], idx_v)

    def add_offsets(p, carry):
        pbase = p * _PERIOD
        for v in range(_PVREGS):
            s = pl.ds(pbase + v * 16, 16)
            idx_v[s] = idx_v[s] + offs_v[pl.ds(v * 16, 16)]
        return carry

    lax.fori_loop(0, _NPERIOD, add_offsets, 0)

    def start_gather(c):
        buf = c % _NBUF
        return pltpu.async_copy(
            table_hbm.at[idx_v.at[pl.ds(c * _CHUNK, _CHUNK)]],
            rows_v.at[buf],
            gsem.at[buf],
        )

    def start_store(c):
        buf = c % _NBUF
        return pltpu.async_copy(
            rows_v.at[buf],
            out_hbm.at[pl.ds(base + c * _CHUNK, _CHUNK)],
            ssem.at[buf],
        )

    gathers = [None] * _NCHUNK
    stores = [None] * _NCHUNK
    gathers[0] = start_gather(0)
    for c in range(_NCHUNK):
        gathers[c].wait()
        if c >= 1:
            stores[c - 1].wait()
        stores[c] = start_store(c)
        if c + 1 < _NCHUNK:
            gathers[c + 1] = start_gather(c + 1)
    stores[_NCHUNK - 1].wait()


@jax.jit
def kernel(x, table):
    mesh = plsc.VectorSubcoreMesh(core_axis_name="c", subcore_axis_name="s")
    run = pl.kernel(
        _sc_body,
        mesh=mesh,
        out_type=jax.ShapeDtypeStruct((_B, _EMBED), jnp.float32),
        scratch_types=[
            pltpu.VMEM((_BPW,), jnp.int32),
            pltpu.VMEM((_PERIOD,), jnp.int32),
            pltpu.VMEM((_NBUF, _CHUNK, _EMBED), jnp.float32),
            pltpu.SemaphoreType.DMA((_NBUF,)),
            pltpu.SemaphoreType.DMA((_NBUF,)),
        ],
        compiler_params=pltpu.CompilerParams(use_tc_tiling_on_sc=False),
    )
    out = run(x.reshape(_B), jnp.asarray(_OFFS_TILED), table)
    return out.reshape(_BATCH, _NUM_FIELDS, _EMBED)


# speedup vs baseline: 1.0022x; 1.0022x over previous
import numpy as np
import jax
import jax.numpy as jnp
from jax import lax
from jax.experimental import pallas as pl
from jax.experimental.pallas import tpu as pltpu
from jax.experimental.pallas import tpu_sc as plsc

_NUM_FIELDS = 26
_VOCAB = 100000
_EMBED = 16
_BATCH = 16384
_B = _BATCH * _NUM_FIELDS
_NC = 2
_NS = 16
_NW = _NC * _NS
_BPW = _B // _NW
_CHUNK = 832
_NCHUNK = _BPW // _CHUNK
_NBUF = 4
_PERIOD = 208
_PVREGS = _PERIOD // 16
_NPERIOD = _BPW // _PERIOD

_OFFS_TILED = np.tile(
    np.array((0, *np.cumsum([_VOCAB] * _NUM_FIELDS)[:-1]), dtype=np.int32),
    _PERIOD // _NUM_FIELDS,
)


def _sc_body(x_hbm, offs_hbm, table_hbm, out_hbm, idx_v, offs_v, rows_v,
             gsem, ssem):
    wid = lax.axis_index("s") * _NC + lax.axis_index("c")
    base = wid * _BPW

    pltpu.sync_copy(offs_hbm, offs_v)
    pltpu.sync_copy(x_hbm.at[pl.ds(base, _BPW)], idx_v)

    def add_offsets(p, carry):
        pbase = p * _PERIOD
        for v in range(_PVREGS):
            s = pl.ds(pbase + v * 16, 16)
            idx_v[s] = idx_v[s] + offs_v[pl.ds(v * 16, 16)]
        return carry

    lax.fori_loop(0, _NPERIOD, add_offsets, 0)

    def start_gather(c):
        buf = c % _NBUF
        return pltpu.async_copy(
            table_hbm.at[idx_v.at[pl.ds(c * _CHUNK, _CHUNK)]],
            rows_v.at[buf],
            gsem.at[buf],
        )

    def start_store(c):
        buf = c % _NBUF
        return pltpu.async_copy(
            rows_v.at[buf],
            out_hbm.at[pl.ds(base + c * _CHUNK, _CHUNK)],
            ssem.at[buf],
        )

    gathers = [None] * _NCHUNK
    stores = [None] * _NCHUNK
    for c in range(_NBUF - 1):
        gathers[c] = start_gather(c)
    for c in range(_NCHUNK):
        gathers[c].wait()
        stores[c] = start_store(c)
        nxt = c + _NBUF - 1
        if nxt < _NCHUNK:
            if c >= 1:
                stores[c - 1].wait()
            gathers[nxt] = start_gather(nxt)
    for c in range(max(0, _NCHUNK - _NBUF), _NCHUNK):
        if stores[c] is not None:
            stores[c].wait()


@jax.jit
def kernel(x, table):
    mesh = plsc.VectorSubcoreMesh(core_axis_name="c", subcore_axis_name="s")
    run = pl.kernel(
        _sc_body,
        mesh=mesh,
        out_type=jax.ShapeDtypeStruct((_B, _EMBED), jnp.float32),
        scratch_types=[
            pltpu.VMEM((_BPW,), jnp.int32),
            pltpu.VMEM((_PERIOD,), jnp.int32),
            pltpu.VMEM((_NBUF, _CHUNK, _EMBED), jnp.float32),
            pltpu.SemaphoreType.DMA((_NBUF,)),
            pltpu.SemaphoreType.DMA((_NBUF,)),
        ],
        compiler_params=pltpu.CompilerParams(use_tc_tiling_on_sc=False),
    )
    out = run(x.reshape(_B), jnp.asarray(_OFFS_TILED), table)
    return out.reshape(_BATCH, _NUM_FIELDS, _EMBED)


# speedup vs baseline: 7.6334x; 7.6165x over previous
import jax
import jax.numpy as jnp
from jax import lax
from jax.experimental import pallas as pl
from jax.experimental.pallas import tpu as pltpu
from jax.experimental.pallas import tpu_sc as plsc

_NUM_FIELDS = 26
_VOCAB = 100000
_EMBED = 16
_BATCH = 16384
_NC = 2
_NS = 16
_FPC = _NUM_FIELDS // _NC
_NB = 8192
_NCHUNK = _BATCH // _NB
_SUBW = 100224
_LAST_F = _NUM_FIELDS - 1
_LAST_C0A = (_LAST_F * _VOCAB // 128) * 128
_LAST_W = 2600000 - _LAST_C0A


def _sc_body(xlin_hbm, tt_hbm, tail_hbm, out_hbm, sub_v, xb_v, ob_v):
    cid = lax.axis_index("c")
    sid = lax.axis_index("s")

    for k in range(_FPC):
        f = cid * _FPC + k
        c0 = f * _VOCAB
        dc = (f % 4) * 32
        c0a = pl.multiple_of(c0 - dc, 128)
        is_last = (cid == 1) & (k == _FPC - 1)

        @pl.when(jnp.logical_not(is_last))
        def _():
            pltpu.sync_copy(tt_hbm.at[sid, pl.ds(c0a, _SUBW)], sub_v)

        @pl.when(is_last)
        def _():
            pltpu.sync_copy(tail_hbm.at[sid, pl.ds(0, _SUBW)], sub_v)

        dcv = jnp.full((16,), dc, jnp.int32)

        for cc in range(_NCHUNK):
            b0 = cc * _NB
            pltpu.sync_copy(
                xlin_hbm.at[pl.ds(pl.multiple_of(f * _BATCH + b0, 1024),
                                  _NB)], xb_v)

            def chunk_body(j, carry):
                base = j * 128
                for u in range(8):
                    s = pl.ds(base + u * 16, 16)
                    ob_v[s] = plsc.load_gather(sub_v, [xb_v[s] + dcv])
                return carry

            lax.fori_loop(0, _NB // 128, chunk_body, 0)
            pltpu.sync_copy(
                ob_v,
                out_hbm.at[pl.ds(f * (_EMBED * _BATCH) + sid * _BATCH + b0,
                                 _NB)])


@jax.jit
def kernel(x, table):
    mesh = plsc.VectorSubcoreMesh(core_axis_name="c", subcore_axis_name="s")
    run = pl.kernel(
        _sc_body,
        mesh=mesh,
        out_type=jax.ShapeDtypeStruct((_NUM_FIELDS * _EMBED * _BATCH,),
                                      jnp.float32),
        scratch_types=[
            pltpu.VMEM((_SUBW,), jnp.float32),
            pltpu.VMEM((_NB,), jnp.int32),
            pltpu.VMEM((_NB,), jnp.float32),
        ],
        compiler_params=pltpu.CompilerParams(needs_layout_passes=False),
    )
    tt = table.T
    tail = jnp.pad(tt[:, _LAST_C0A:], ((0, 0), (0, _SUBW - _LAST_W)))
    out = run(x.T.reshape(_NUM_FIELDS * _BATCH), tt, tail)
    return jnp.transpose(out.reshape(_NUM_FIELDS, _EMBED, _BATCH), (2, 0, 1))


# speedup vs baseline: 7.9037x; 1.0354x over previous
import numpy as np
import jax
import jax.numpy as jnp
from jax import lax
from jax.experimental import pallas as pl
from jax.experimental.pallas import tpu as pltpu
from jax.experimental.pallas import tpu_sc as plsc

_NUM_FIELDS = 26
_VOCAB = 100000
_EMBED = 16
_BATCH = 16384
_NC = 2
_NS = 16
_FPC = _NUM_FIELDS // _NC
_NB = 4096
_NCHUNK = _BATCH // _NB
_FW = 100096
_FH = _FW // 2
_SUBW = 100224
_LAST_C0A = (25 * _VOCAB // 128) * 128
_LAST_W = 2600000 - _LAST_C0A

_DELTAS = np.array([(f % 4) * 32 for f in range(_NUM_FIELDS)], np.int32)


def _sc_body(xlin_hbm, tt_hbm, tail_hbm, out_hbm, sub_v, xb_v, ob_v,
             rsem, xsem, osem):
    cid = lax.axis_index("c")
    sid = lax.axis_index("s")

    def xb_start(f, cc, buf):
        return pltpu.async_copy(
            xlin_hbm.at[pl.ds(f * _BATCH + cc * _NB, _NB)],
            xb_v.at[buf], xsem.at[buf])

    xh = [None, None]
    oh = [None, None]
    xh[0] = xb_start(cid * _FPC, 0, 0)

    for k in range(_FPC):
        f = cid * _FPC + k
        c0a = pl.multiple_of(f * _VOCAB - (f % 4) * 32, 128)

        rh = []
        if k < _FPC - 1:
            rh.append(pltpu.async_copy(
                tt_hbm.at[sid, pl.ds(c0a, _FH)],
                sub_v.at[pl.ds(0, _FH)], rsem.at[0]))
            rh.append(pltpu.async_copy(
                tt_hbm.at[sid, pl.ds(c0a + _FH, _FH)],
                sub_v.at[pl.ds(_FH, _FH)], rsem.at[1]))
        else:
            @pl.when(cid == 0)
            def _():
                pltpu.sync_copy(tt_hbm.at[sid, pl.ds(c0a, _FW)],
                                sub_v.at[pl.ds(0, _FW)])

            @pl.when(cid == 1)
            def _():
                pltpu.sync_copy(tail_hbm.at[sid, pl.ds(0, _SUBW)], sub_v)

        for cc in range(_NCHUNK):
            buf = cc & 1
            nbuf = (cc + 1) & 1
            if cc + 1 < _NCHUNK:
                xh[nbuf] = xb_start(f, cc + 1, nbuf)
            elif k + 1 < _FPC:
                xh[nbuf] = xb_start(f + 1, 0, nbuf)

            if cc == 0:
                for h in rh:
                    h.wait()
            xh[buf].wait()
            if oh[buf] is not None:
                oh[buf].wait()

            def chunk_body(j, carry):
                base = j * 256
                for u in range(16):
                    s = pl.ds(base + u * 16, 16)
                    ob_v[buf, s] = plsc.load_gather(sub_v, [xb_v[buf, s]])
                return carry

            lax.fori_loop(0, _NB // 256, chunk_body, 0)
            oh[buf] = pltpu.async_copy(
                ob_v.at[buf],
                out_hbm.at[pl.ds(f * (_EMBED * _BATCH) + sid * _BATCH
                                 + cc * _NB, _NB)],
                osem.at[buf])

    for h in oh:
        if h is not None:
            h.wait()


@jax.jit
def kernel(x, table):
    mesh = plsc.VectorSubcoreMesh(core_axis_name="c", subcore_axis_name="s")
    run = pl.kernel(
        _sc_body,
        mesh=mesh,
        out_type=jax.ShapeDtypeStruct((_NUM_FIELDS * _EMBED * _BATCH,),
                                      jnp.float32),
        scratch_types=[
            pltpu.VMEM((_SUBW,), jnp.float32),
            pltpu.VMEM((2, _NB), jnp.int32),
            pltpu.VMEM((2, _NB), jnp.float32),
            pltpu.SemaphoreType.DMA((2,)),
            pltpu.SemaphoreType.DMA((2,)),
            pltpu.SemaphoreType.DMA((2,)),
        ],
        compiler_params=pltpu.CompilerParams(needs_layout_passes=False),
    )
    tt = table.T
    tail = jnp.pad(tt[:, _LAST_C0A:], ((0, 0), (0, _SUBW - _LAST_W)))
    xlin = (x + _DELTAS[None, :]).T.reshape(_NUM_FIELDS * _BATCH)
    out = run(xlin, tt, tail)
    return jnp.transpose(out.reshape(_NUM_FIELDS, _EMBED, _BATCH), (2, 0, 1))
